# baseline (speedup 1.0000x reference)
import functools

import numpy as np
import jax
import jax.numpy as jnp
from jax import lax
from jax.experimental import pallas as pl
from jax.experimental.pallas import tpu as pltpu
from jax.experimental.pallas import tpu_sc as plsc

EMBED = 64
SEQ = 50
NUM_CORES = 2
NUM_SUBCORES = 16
NUM_WORKERS = NUM_CORES * NUM_SUBCORES
RPC = 2
NBUF = 4
LANES = 16
VREGS_PER_ROW = EMBED // LANES


def _positional_encoding(seq_len, d_model):
    pos = np.arange(seq_len)[:, np.newaxis]
    i = np.arange(d_model)[np.newaxis, :]
    angle_rates = 1.0 / np.power(10000, 2 * (i // 2) / np.float32(d_model))
    angle_rads = pos * angle_rates
    angle_rads[:, 0::2] = np.sin(angle_rads[:, 0::2])
    angle_rads[:, 1::2] = np.cos(angle_rads[:, 1::2])
    return angle_rads.astype(np.float32)


TW = 320


@functools.lru_cache(maxsize=None)
def _build_transpose(vocab):
    total_chunks = vocab // TW
    iters = (total_chunks + NUM_WORKERS - 1) // NUM_WORKERS

    @functools.partial(
        pl.kernel,
        mesh=plsc.VectorSubcoreMesh(core_axis_name="c", subcore_axis_name="s"),
        out_type=jax.ShapeDtypeStruct((vocab * EMBED,), jnp.float32),
        scratch_types=(
            [pltpu.VMEM((EMBED, TW), jnp.float32)] * 2
            + [pltpu.VMEM((TW * EMBED,), jnp.float32)] * 2
            + [pltpu.SemaphoreType.DMA] * 2
        ),
        compiler_params=pltpu.CompilerParams(
            use_tc_tiling_on_sc=False, needs_layout_passes=False),
    )
    def tr_kernel(tt_hbm, out_hbm, f0, f1, r0, r1, g0, g1):
        w = lax.axis_index("s") * NUM_CORES + lax.axis_index("c")
        fbufs, rbufs, gsems = (f0, f1), (r0, r1), (g0, g1)
        iota = lax.iota(jnp.int32, LANES)

        def in_copy(ci, s):
            return pltpu.make_async_copy(
                tt_hbm.at[:, pl.ds(ci * TW, TW)], fbufs[s], gsems[s])

        in_copy(w, 0).start()

        cvecs = [iota + (16 * m) for m in range(VREGS_PER_ROW)]

        def do_chunk(ci, s):
            @pl.when(ci < total_chunks)
            def _():
                in_copy(ci, s).wait()

                @pl.when(ci + NUM_WORKERS < total_chunks)
                def _():
                    in_copy(ci + NUM_WORKERS, 1 - s).start()

                def tr_rows(v, carry):
                    vvec = jnp.full((LANES,), v, jnp.int32)
                    for m in range(VREGS_PER_ROW):
                        rbufs[s][pl.ds(v * EMBED + 16 * m, LANES)] = (
                            plsc.load_gather(fbufs[s], [cvecs[m], vvec]))
                    return carry

                lax.fori_loop(0, TW, tr_rows, 0)
                pltpu.sync_copy(rbufs[s],
                                out_hbm.at[pl.ds(ci * TW * EMBED, TW * EMBED)])

        def step(t2, carry):
            for s in range(2):
                do_chunk(w + NUM_WORKERS * (2 * t2 + s), s)
            return carry

        lax.fori_loop(0, (iters + 1) // 2, step, 0)

    return tr_kernel


@functools.lru_cache(maxsize=None)
def _build(batch, vocab):
    bpw = batch // NUM_WORKERS
    num_chunks = bpw // RPC

    @functools.partial(
        pl.kernel,
        mesh=plsc.VectorSubcoreMesh(core_axis_name="c", subcore_axis_name="s"),
        out_type=jax.ShapeDtypeStruct((batch, SEQ, EMBED), jnp.float32),
        scratch_types=(
            [pltpu.VMEM((bpw, SEQ), jnp.int32),
             pltpu.VMEM((SEQ, EMBED), jnp.float32)]
            + [pltpu.VMEM((RPC, SEQ, EMBED), jnp.float32)] * NBUF
            + [pltpu.SemaphoreType.DMA] * (2 * NBUF)
        ),
        compiler_params=pltpu.CompilerParams(use_tc_tiling_on_sc=False),
    )
    def emb_kernel(x_hbm, pe_hbm, table_hbm, out_hbm, idx_v, pe_v, *bufsem):
        bufs = bufsem[:NBUF]
        gsem = bufsem[NBUF:2 * NBUF]
        ssem = bufsem[2 * NBUF:]
        w = lax.axis_index("s") * NUM_CORES + lax.axis_index("c")
        batch_base = w * bpw
        pltpu.sync_copy(x_hbm.at[pl.ds(batch_base, bpw)], idx_v)
        pltpu.sync_copy(pe_hbm, pe_v)

        def gather(j, s):
            for k in range(RPC):
                pltpu.async_copy(
                    table_hbm.at[idx_v.at[RPC * j + k]], bufs[s].at[k],
                    gsem[s])

        def store_copy(j, s):
            return pltpu.make_async_copy(
                bufs[s],
                out_hbm.at[pl.ds(batch_base + RPC * j, RPC)], ssem[s])

        for s in range(NBUF - 1):
            gather(s, s)

        def process(j, s, sp):
            for k in range(RPC):
                pltpu.make_async_copy(
                    table_hbm.at[idx_v.at[RPC * j + k]], bufs[s].at[k],
                    gsem[s]).wait()

            def add_rows(i, carry):
                r0 = i * 2
                for k in range(RPC):
                    for r in range(2):
                        for d in range(VREGS_PER_ROW):
                            sl = pl.ds(d * LANES, LANES)
                            bufs[s][k, r0 + r, sl] = (
                                bufs[s][k, r0 + r, sl] + pe_v[r0 + r, sl])
                return carry

            lax.fori_loop(0, SEQ // 2, add_rows, 0)
            store_copy(j, s).start()

            @pl.when(j + NBUF - 1 < num_chunks)
            def _():
                @pl.when(j >= 1)
                def _():
                    store_copy(j - 1, sp).wait()
                gather(j + NBUF - 1, sp)

        def step(t, carry):
            for s in range(NBUF):
                j = NBUF * t + s
                process(j, s, (s + NBUF - 1) % NBUF)
            return carry

        lax.fori_loop(0, num_chunks // NBUF, step, 0)
        for s in range(NBUF):
            store_copy(num_chunks - NBUF + s, s).wait()

    return emb_kernel


def kernel(x, table):
    batch, seq = x.shape
    vocab, embed = table.shape
    assert embed == EMBED and seq == SEQ
    assert batch % (NUM_WORKERS * RPC * NBUF) == 0
    pe = _positional_encoding(SEQ, EMBED)
    table_flat = _build_transpose(vocab)(jnp.transpose(table))
    table_rm = table_flat.reshape(vocab, embed)
    return _build(batch, vocab)(x, jnp.asarray(pe), table_rm)


# speedup vs baseline: 5.7264x; 5.7264x over previous
import functools

import numpy as np
import jax
import jax.numpy as jnp
from jax import lax
from jax.experimental import pallas as pl
from jax.experimental.pallas import tpu as pltpu
from jax.experimental.pallas import tpu_sc as plsc

EMBED = 64
SEQ = 50
NUM_CORES = 2
NUM_SUBCORES = 16
NUM_WORKERS = NUM_CORES * NUM_SUBCORES
RPC = 4
NBUF = 4
LANES = 16
VREGS_PER_ROW = EMBED // LANES


def _positional_encoding(seq_len, d_model):
    pos = np.arange(seq_len)[:, np.newaxis]
    i = np.arange(d_model)[np.newaxis, :]
    angle_rates = 1.0 / np.power(10000, 2 * (i // 2) / np.float32(d_model))
    angle_rads = pos * angle_rates
    angle_rads[:, 0::2] = np.sin(angle_rads[:, 0::2])
    angle_rads[:, 1::2] = np.cos(angle_rads[:, 1::2])
    return angle_rads.astype(np.float32)


@functools.lru_cache(maxsize=None)
def _build(batch, vocab):
    bpw = batch // NUM_WORKERS
    num_chunks = bpw // RPC

    @functools.partial(
        pl.kernel,
        mesh=plsc.VectorSubcoreMesh(core_axis_name="c", subcore_axis_name="s"),
        out_type=jax.ShapeDtypeStruct((batch, SEQ, EMBED), jnp.float32),
        scratch_types=(
            [pltpu.VMEM((bpw, SEQ), jnp.int32),
             pltpu.VMEM((SEQ, EMBED), jnp.float32)]
            + [pltpu.VMEM((RPC, SEQ, EMBED), jnp.float32)] * NBUF
            + [pltpu.SemaphoreType.DMA] * (2 * NBUF)
        ),
        compiler_params=pltpu.CompilerParams(use_tc_tiling_on_sc=False),
    )
    def emb_kernel(x_hbm, pe_hbm, table_hbm, out_hbm, idx_v, pe_v, *bufsem):
        bufs = bufsem[:NBUF]
        gsem = bufsem[NBUF:2 * NBUF]
        ssem = bufsem[2 * NBUF:]
        w = lax.axis_index("s") * NUM_CORES + lax.axis_index("c")
        batch_base = w * bpw
        pltpu.sync_copy(x_hbm.at[pl.ds(batch_base, bpw)], idx_v)
        pltpu.sync_copy(pe_hbm, pe_v)

        def gather(j, s):
            for k in range(RPC):
                pltpu.async_copy(
                    table_hbm.at[idx_v.at[RPC * j + k]], bufs[s].at[k],
                    gsem[s])

        def store_copy(j, s):
            return pltpu.make_async_copy(
                bufs[s],
                out_hbm.at[pl.ds(batch_base + RPC * j, RPC)], ssem[s])

        for s in range(NBUF - 1):
            gather(s, s)

        def process(j, s, sp):
            for k in range(RPC):
                pltpu.make_async_copy(
                    table_hbm.at[idx_v.at[RPC * j + k]], bufs[s].at[k],
                    gsem[s]).wait()

            def add_rows(i, carry):
                r0 = i * 2
                for k in range(RPC):
                    for r in range(2):
                        for d in range(VREGS_PER_ROW):
                            sl = pl.ds(d * LANES, LANES)
                            bufs[s][k, r0 + r, sl] = (
                                bufs[s][k, r0 + r, sl] + pe_v[r0 + r, sl])
                return carry

            lax.fori_loop(0, SEQ // 2, add_rows, 0)
            store_copy(j, s).start()

            @pl.when(j + NBUF - 1 < num_chunks)
            def _():
                @pl.when(j >= 1)
                def _():
                    store_copy(j - 1, sp).wait()
                gather(j + NBUF - 1, sp)

        def step(t, carry):
            for s in range(NBUF):
                j = NBUF * t + s
                process(j, s, (s + NBUF - 1) % NBUF)
            return carry

        lax.fori_loop(0, num_chunks // NBUF, step, 0)
        for s in range(NBUF):
            store_copy(num_chunks - NBUF + s, s).wait()

    return emb_kernel


def kernel(x, table):
    batch, seq = x.shape
    vocab, embed = table.shape
    assert embed == EMBED and seq == SEQ
    assert batch % (NUM_WORKERS * RPC * NBUF) == 0
    pe = _positional_encoding(SEQ, EMBED)
    return _build(batch, vocab)(x, jnp.asarray(pe), table)


# speedup vs baseline: 5.7773x; 1.0089x over previous
import functools

import numpy as np
import jax
import jax.numpy as jnp
from jax import lax
from jax.experimental import pallas as pl
from jax.experimental.pallas import tpu as pltpu
from jax.experimental.pallas import tpu_sc as plsc

EMBED = 64
SEQ = 50
NUM_CORES = 2
NUM_SUBCORES = 16
NUM_WORKERS = NUM_CORES * NUM_SUBCORES
RPC = 4
NBUF = 4
LANES = 16
VREGS_PER_ROW = EMBED // LANES


def _positional_encoding(seq_len, d_model):
    pos = np.arange(seq_len)[:, np.newaxis]
    i = np.arange(d_model)[np.newaxis, :]
    angle_rates = 1.0 / np.power(10000, 2 * (i // 2) / np.float32(d_model))
    angle_rads = pos * angle_rates
    angle_rads[:, 0::2] = np.sin(angle_rads[:, 0::2])
    angle_rads[:, 1::2] = np.cos(angle_rads[:, 1::2])
    return angle_rads.astype(np.float32)


@functools.lru_cache(maxsize=None)
def _build(batch, vocab):
    bpw = batch // NUM_WORKERS
    num_chunks = bpw // RPC

    @functools.partial(
        pl.kernel,
        mesh=plsc.VectorSubcoreMesh(core_axis_name="c", subcore_axis_name="s"),
        out_type=jax.ShapeDtypeStruct((batch, SEQ, EMBED), jnp.float32),
        scratch_types=(
            [pltpu.VMEM((bpw, SEQ), jnp.int32),
             pltpu.VMEM((SEQ, EMBED), jnp.float32)]
            + [pltpu.VMEM((RPC, SEQ, EMBED), jnp.float32)] * NBUF
            + [pltpu.SemaphoreType.DMA] * (2 * NBUF)
        ),
        compiler_params=pltpu.CompilerParams(use_tc_tiling_on_sc=False),
    )
    def emb_kernel(x_hbm, pe_hbm, table_hbm, out_hbm, idx_v, pe_v, *bufsem):
        bufs = bufsem[:NBUF]
        gsem = bufsem[NBUF:2 * NBUF]
        ssem = bufsem[2 * NBUF:]
        w = lax.axis_index("s") * NUM_CORES + lax.axis_index("c")
        batch_base = w * bpw
        pltpu.sync_copy(x_hbm.at[pl.ds(batch_base, bpw)], idx_v)
        pltpu.sync_copy(pe_hbm, pe_v)

        def gather(j, s):
            for k in range(RPC):
                pltpu.async_copy(
                    table_hbm.at[idx_v.at[RPC * j + k]], bufs[s].at[k],
                    gsem[s])

        def store_copy(j, s):
            return pltpu.make_async_copy(
                bufs[s],
                out_hbm.at[pl.ds(batch_base + RPC * j, RPC)], ssem[s])

        for s in range(NBUF - 1):
            gather(s, s)

        def process(j, s, sp):
            for k in range(RPC):
                pltpu.make_async_copy(
                    table_hbm.at[idx_v.at[RPC * j + k]], bufs[s].at[k],
                    gsem[s]).wait()

            def add_rows(r, carry):
                for d in range(VREGS_PER_ROW):
                    sl = pl.ds(d * LANES, LANES)
                    pv = pe_v[r, sl]
                    for k in range(RPC):
                        bufs[s][k, r, sl] = bufs[s][k, r, sl] + pv
                return carry

            lax.fori_loop(0, SEQ, add_rows, 0)
            store_copy(j, s).start()

            @pl.when(j + NBUF - 1 < num_chunks)
            def _():
                @pl.when(j >= 1)
                def _():
                    store_copy(j - 1, sp).wait()
                gather(j + NBUF - 1, sp)

        def step(t, carry):
            for s in range(NBUF):
                j = NBUF * t + s
                process(j, s, (s + NBUF - 1) % NBUF)
            return carry

        lax.fori_loop(0, num_chunks // NBUF, step, 0)
        for s in range(NBUF):
            store_copy(num_chunks - NBUF + s, s).wait()

    return emb_kernel


def kernel(x, table):
    batch, seq = x.shape
    vocab, embed = table.shape
    assert embed == EMBED and seq == SEQ
    assert batch % (NUM_WORKERS * RPC * NBUF) == 0
    pe = _positional_encoding(SEQ, EMBED)
    return _build(batch, vocab)(x, jnp.asarray(pe), table)
